# baseline (speedup 1.0000x reference)
import functools

import jax
import jax.numpy as jnp
from jax import lax
from jax.experimental import pallas as pl
from jax.experimental.pallas import tpu as pltpu
from jax.experimental.pallas import tpu_sc as plsc

B = 16384
F = 26
V = 100000
D = 32
NUM = 13
H = 128

FA = 16
FB = F - FA

A = 24960
NJ = 13
AJ = A // NJ
TV = V - 4 * A
PR = (NJ + 1) * AJ

NW = 32
BPW = B // NW
CH = 128
G = 4


def _transpose_tables(tabT, f0, fh):
    def body(t_ref, e_ref, o_ref):
        j = pl.program_id(1)
        for jj in range(NJ):

            @pl.when(j == jj)
            def _():
                stacked = jnp.concatenate(
                    [
                        t_ref[0, :, s * A + jj * AJ : s * A + (jj + 1) * AJ]
                        for s in range(4)
                    ],
                    axis=0,
                )
                o_ref[...] = lax.dot_general(
                    stacked,
                    e_ref[...],
                    (((0,), (0,)), ((), ())),
                    preferred_element_type=jnp.float32,
                )

        @pl.when(j == NJ)
        def _():
            tail = lax.dot_general(
                t_ref[0, :, 4 * A : V],
                e_ref[0:D, :],
                (((0,), (0,)), ((), ())),
                preferred_element_type=jnp.float32,
            )
            o_ref[0:TV, :] = tail

    return pl.pallas_call(
        body,
        grid=(fh, NJ + 1),
        in_specs=[
            pl.BlockSpec((1, D, V), lambda f, j: (f + f0, 0, 0)),
            pl.BlockSpec((4 * D, 4 * D), lambda f, j: (0, 0)),
        ],
        out_specs=pl.BlockSpec((AJ, 4 * D), lambda f, j: (f * (NJ + 1) + j, 0)),
        out_shape=jax.ShapeDtypeStruct((fh * (NJ + 1) * AJ, 4 * D), jnp.float32),
    )(tabT, jnp.eye(4 * D, dtype=jnp.float32))


def _sc_gather(cat3, tab2, fh):
    NCH = BPW * fh // CH
    NG = NCH // G
    mesh = plsc.VectorSubcoreMesh(core_axis_name="c", subcore_axis_name="s")

    @functools.partial(
        pl.kernel,
        mesh=mesh,
        compiler_params=pltpu.CompilerParams(use_tc_tiling_on_sc=False),
        out_type=jax.ShapeDtypeStruct((NW, NG, G, CH, D), jnp.float32),
        scratch_types=[
            pltpu.VMEM((NCH, CH), jnp.int32),
            pltpu.VMEM((G, CH, D), jnp.float32),
            pltpu.VMEM((G, CH, D), jnp.float32),
            pltpu.SemaphoreType.DMA,
            pltpu.SemaphoreType.DMA,
        ],
    )
    def k(cat_h, tab_h, out_h, idx_v, buf0, buf1, sem0, sem1):
        wid = lax.axis_index("s") * 2 + lax.axis_index("c")
        pltpu.sync_copy(cat_h.at[wid], idx_v)

        def fire(g, buf, sem):
            return [
                pltpu.async_copy(tab_h.at[idx_v.at[g * G + j]], buf.at[j], sem)
                for j in range(G)
            ]

        def group_body(i, carry):
            g0 = i * 2
            g1 = g0 + 1
            cps0 = fire(g0, buf0, sem0)
            cps1 = fire(g1, buf1, sem1)
            for cp in cps0:
                cp.wait()
            pltpu.sync_copy(buf0, out_h.at[wid, g0])
            for cp in cps1:
                cp.wait()
            pltpu.sync_copy(buf1, out_h.at[wid, g1])
            return carry

        lax.fori_loop(0, NG // 2, group_body, 0)

    return k(cat3, tab2)


def _half_indices(v, fh):
    s = v // A
    offs = jnp.arange(fh, dtype=jnp.int32) * PR
    idx = jnp.where(
        s < 4,
        4 * (offs[None, :] + v - s * A) + s,
        4 * (offs[None, :] + v - 3 * A),
    )
    return idx.reshape(NW, BPW * fh // CH, CH)


def _mlp(e1, e2, num, w1a1, w1a2, w1b, b1r, w2r, b2r):
    BLK = 1024
    EA = FA * D
    EB = FB * D

    def body(e1_ref, e2_ref, n_ref, wa1_ref, wa2_ref, wb_ref, b1_ref,
             w2_ref, b2_ref, o_ref):
        x = jnp.dot(e1_ref[...], wa1_ref[...], preferred_element_type=jnp.float32)
        x = x + jnp.dot(e2_ref[...], wa2_ref[...], preferred_element_type=jnp.float32)
        x = x + jnp.dot(n_ref[...], wb_ref[...], preferred_element_type=jnp.float32)
        x = jnp.maximum(x + b1_ref[...], 0.0)
        y = jnp.sum(x * w2_ref[...], axis=1, keepdims=True) + b2_ref[...]
        o_ref[...] = jax.nn.sigmoid(y)

    return pl.pallas_call(
        body,
        grid=(B // BLK,),
        in_specs=[
            pl.BlockSpec((BLK, EA), lambda i: (i, 0)),
            pl.BlockSpec((BLK, EB), lambda i: (i, 0)),
            pl.BlockSpec((BLK, NUM), lambda i: (i, 0)),
            pl.BlockSpec((EA, H), lambda i: (0, 0)),
            pl.BlockSpec((EB, H), lambda i: (0, 0)),
            pl.BlockSpec((NUM, H), lambda i: (0, 0)),
            pl.BlockSpec((1, H), lambda i: (0, 0)),
            pl.BlockSpec((1, H), lambda i: (0, 0)),
            pl.BlockSpec((1, 1), lambda i: (0, 0)),
        ],
        out_specs=pl.BlockSpec((BLK, 1), lambda i: (i, 0)),
        out_shape=jax.ShapeDtypeStruct((B, 1), jnp.float32),
    )(e1, e2, num, w1a1, w1a2, w1b, b1r, w2r, b2r)


def kernel(categorical_inputs, numerical_inputs, tables, W1, b1, W2, b2):
    tabT = tables.transpose(0, 2, 1)
    v = categorical_inputs.astype(jnp.int32)

    halves = []
    for f0, fh in ((0, FA), (FA, FB)):
        tabP = _transpose_tables(tabT, f0, fh)
        tab2 = tabP.reshape(fh * PR * 4, D)
        cat3 = _half_indices(v[:, f0 : f0 + fh], fh)
        embs5 = _sc_gather(cat3, tab2, fh)
        halves.append(embs5.reshape(B, fh * D))

    w1a1 = W1[: FA * D]
    w1a2 = W1[FA * D : F * D]
    w1b = W1[F * D :]
    return _mlp(
        halves[0],
        halves[1],
        numerical_inputs,
        w1a1,
        w1a2,
        w1b,
        b1.reshape(1, H),
        W2.reshape(1, H),
        b2.reshape(1, 1),
    )


# speedup vs baseline: 1.2179x; 1.2179x over previous
import functools

import jax
import jax.numpy as jnp
from jax import lax
from jax.experimental import pallas as pl
from jax.experimental.pallas import tpu as pltpu
from jax.experimental.pallas import tpu_sc as plsc

B = 16384
F = 26
V = 100000
D = 32
NUM = 13
H = 128

FA = 16
FB = F - FA

A = 24960
NJ = 5
AJ = A // NJ
TV = V - 4 * A
PR = (NJ + 1) * AJ

NW = 32
BPW = B // NW
CH = 128
G = 4


def _transpose_tables(tabT, f0, fh):
    def body(t_ref, e_ref, o_ref):
        j = pl.program_id(1)
        for jj in range(NJ):

            @pl.when(j == jj)
            def _():
                stacked = jnp.concatenate(
                    [
                        t_ref[0, :, s * A + jj * AJ : s * A + (jj + 1) * AJ]
                        for s in range(4)
                    ],
                    axis=0,
                )
                o_ref[...] = lax.dot_general(
                    stacked,
                    e_ref[...],
                    (((0,), (0,)), ((), ())),
                    preferred_element_type=jnp.float32,
                )

        @pl.when(j == NJ)
        def _():
            tail = lax.dot_general(
                t_ref[0, :, 4 * A : V],
                e_ref[0:D, :],
                (((0,), (0,)), ((), ())),
                preferred_element_type=jnp.float32,
            )
            o_ref[0:TV, :] = tail

    return pl.pallas_call(
        body,
        grid=(fh, NJ + 1),
        in_specs=[
            pl.BlockSpec((1, D, V), lambda f, j: (f + f0, 0, 0)),
            pl.BlockSpec((4 * D, 4 * D), lambda f, j: (0, 0)),
        ],
        out_specs=pl.BlockSpec((AJ, 4 * D), lambda f, j: (f * (NJ + 1) + j, 0)),
        out_shape=jax.ShapeDtypeStruct((fh * (NJ + 1) * AJ, 4 * D), jnp.float32),
    )(tabT, jnp.eye(4 * D, dtype=jnp.float32))


def _sc_gather(cat3, tab2, fh):
    NCH = BPW * fh // CH
    NG = NCH // G
    mesh = plsc.VectorSubcoreMesh(core_axis_name="c", subcore_axis_name="s")

    @functools.partial(
        pl.kernel,
        mesh=mesh,
        compiler_params=pltpu.CompilerParams(use_tc_tiling_on_sc=False),
        out_type=jax.ShapeDtypeStruct((NW, NG, G, CH, D), jnp.float32),
        scratch_types=[
            pltpu.VMEM((NCH, CH), jnp.int32),
            pltpu.VMEM((G, CH, D), jnp.float32),
            pltpu.VMEM((G, CH, D), jnp.float32),
            pltpu.SemaphoreType.DMA,
            pltpu.SemaphoreType.DMA,
        ],
    )
    def k(cat_h, tab_h, out_h, idx_v, buf0, buf1, sem0, sem1):
        wid = lax.axis_index("s") * 2 + lax.axis_index("c")
        pltpu.sync_copy(cat_h.at[wid], idx_v)

        def fire(g, buf, sem):
            return [
                pltpu.async_copy(tab_h.at[idx_v.at[g * G + j]], buf.at[j], sem)
                for j in range(G)
            ]

        def group_body(i, carry):
            g0 = i * 2
            g1 = g0 + 1
            cps0 = fire(g0, buf0, sem0)
            cps1 = fire(g1, buf1, sem1)
            for cp in cps0:
                cp.wait()
            pltpu.sync_copy(buf0, out_h.at[wid, g0])
            for cp in cps1:
                cp.wait()
            pltpu.sync_copy(buf1, out_h.at[wid, g1])
            return carry

        lax.fori_loop(0, NG // 2, group_body, 0)

    return k(cat3, tab2)


def _half_indices(v, fh):
    s = v // A
    offs = jnp.arange(fh, dtype=jnp.int32) * PR
    idx = jnp.where(
        s < 4,
        4 * (offs[None, :] + v - s * A) + s,
        4 * (offs[None, :] + v - 3 * A),
    )
    return idx.reshape(NW, BPW * fh // CH, CH)


def _mlp(e1, e2, num, w1a1, w1a2, w1b, b1r, w2r, b2r):
    BLK = 1024
    EB = FB * D

    def body(e1_ref, e2_ref, n_ref, wa1_ref, wa2_ref, wb_ref, b1_ref,
             w2_ref, b2_ref, o_ref):
        e3 = e1_ref[...].reshape(BLK, 4, H)
        x = None
        for j in range(4):
            part = jnp.dot(
                e3[:, j, :],
                wa1_ref[j * H : (j + 1) * H, :],
                preferred_element_type=jnp.float32,
            )
            x = part if x is None else x + part
        x = x + jnp.dot(e2_ref[...], wa2_ref[...], preferred_element_type=jnp.float32)
        x = x + jnp.dot(n_ref[...], wb_ref[...], preferred_element_type=jnp.float32)
        x = jnp.maximum(x + b1_ref[...], 0.0)
        y = jnp.sum(x * w2_ref[...], axis=1, keepdims=True) + b2_ref[...]
        o_ref[...] = jax.nn.sigmoid(y)

    return pl.pallas_call(
        body,
        grid=(B // BLK,),
        in_specs=[
            pl.BlockSpec((BLK * 4, H), lambda i: (i, 0)),
            pl.BlockSpec((BLK, EB), lambda i: (i, 0)),
            pl.BlockSpec((BLK, NUM), lambda i: (i, 0)),
            pl.BlockSpec((FA * D, H), lambda i: (0, 0)),
            pl.BlockSpec((EB, H), lambda i: (0, 0)),
            pl.BlockSpec((NUM, H), lambda i: (0, 0)),
            pl.BlockSpec((1, H), lambda i: (0, 0)),
            pl.BlockSpec((1, H), lambda i: (0, 0)),
            pl.BlockSpec((1, 1), lambda i: (0, 0)),
        ],
        out_specs=pl.BlockSpec((BLK, 1), lambda i: (i, 0)),
        out_shape=jax.ShapeDtypeStruct((B, 1), jnp.float32),
    )(e1, e2, num, w1a1, w1a2, w1b, b1r, w2r, b2r)


def kernel(categorical_inputs, numerical_inputs, tables, W1, b1, W2, b2):
    tabT = tables.transpose(0, 2, 1)
    v = categorical_inputs.astype(jnp.int32)

    halves = []
    for f0, fh in ((0, FA), (FA, FB)):
        tabP = _transpose_tables(tabT, f0, fh)
        tab2 = tabP.reshape(fh * PR * 4, D)
        cat3 = _half_indices(v[:, f0 : f0 + fh], fh)
        embs5 = _sc_gather(cat3, tab2, fh)
        halves.append(embs5)
    e1 = halves[0].reshape(B * 4, H)
    e2 = halves[1].reshape(B, FB * D)

    w1a1 = W1[: FA * D]
    w1a2 = W1[FA * D : F * D]
    w1b = W1[F * D :]
    return _mlp(
        e1,
        e2,
        numerical_inputs,
        w1a1,
        w1a2,
        w1b,
        b1.reshape(1, H),
        W2.reshape(1, H),
        b2.reshape(1, 1),
    )
